# baseline (speedup 1.0000x reference)
import functools

import jax
import jax.numpy as jnp
from jax import lax
from jax.experimental import pallas as pl
from jax.experimental.pallas import tpu as pltpu
from jax.experimental.pallas import tpu_sc as plsc

N = 10000
E = 320000
C = 128
A = 10
RB = 8
H = 64
CORR = 3

NC = 2
NS = 16
NW = NC * NS
K = 64
NCHUNK = 158
EPW = K * NCHUNK
E_PAD = EPW * NW
ROWS_PER_TILE = 632
N_PAD = ROWS_PER_TILE * NS

_BLK_N = 1000
_BLK_E = 2048


def _pre_body(nf_ref, w_ref, h_ref):
    h_ref[...] = jnp.dot(nf_ref[...], w_ref[...],
                         preferred_element_type=jnp.float32)


def _scale_body(rb_ref, sph_ref, w1_ref, w2_ref, w3_ref, s_ref):
    x = jax.nn.silu(jnp.dot(rb_ref[...], w1_ref[...],
                            preferred_element_type=jnp.float32))
    x = jax.nn.silu(jnp.dot(x, w2_ref[...],
                            preferred_element_type=jnp.float32))
    w = jnp.dot(x, w3_ref[...], preferred_element_type=jnp.float32)
    s_ref[...] = w * sph_ref[...]


def _post_body(p_ref, nf_ref, attr_ref, wpi_ref, wc_ref, wsct_ref, wp_ref,
               out_ref):
    tm = p_ref[0] + p_ref[1]
    m = jnp.dot(tm, wpi_ref[...], preferred_element_type=jnp.float32)
    attr = attr_ref[...]
    m2 = m * m
    contracted = (jnp.dot(attr, wc_ref[0], preferred_element_type=jnp.float32) * m
                  + jnp.dot(attr, wc_ref[1], preferred_element_type=jnp.float32) * m2
                  + jnp.dot(attr, wc_ref[2], preferred_element_type=jnp.float32) * (m2 * m))
    nf = nf_ref[...]
    sc = jnp.zeros_like(m)
    for a in range(A):
        sc = sc + attr[:, a:a + 1] * jnp.dot(nf, wsct_ref[a],
                                             preferred_element_type=jnp.float32)
    out_ref[...] = jnp.dot(contracted + sc, wp_ref[...],
                           preferred_element_type=jnp.float32)


def _agg_body(h_hbm, s_hbm, src_hbm, dst_hbm, zeros_hbm, out_hbm,
              src_v, dst_v, rows_v, s_v, acc, isem, gsem, ssem, csem):
    cid = lax.axis_index("c")
    sid = lax.axis_index("s")
    wid = sid * NC + cid
    base = wid * EPW
    my_rows = pl.ds(sid * ROWS_PER_TILE, ROWS_PER_TILE)

    def idx_start(k):
        off = base + k * K
        pltpu.async_copy(src_hbm.at[pl.ds(off, K)], src_v.at[k % 4], isem.at[k % 4])
        pltpu.async_copy(dst_hbm.at[pl.ds(off, K)], dst_v.at[k % 4], isem.at[k % 4])

    def idx_wait(k):
        off = base + k * K
        pltpu.make_async_copy(src_hbm.at[pl.ds(off, K)], src_v.at[k % 4], isem.at[k % 4]).wait()
        pltpu.make_async_copy(dst_hbm.at[pl.ds(off, K)], dst_v.at[k % 4], isem.at[k % 4]).wait()

    def main_start(k):
        b = k % 2
        off = base + k * K
        pltpu.async_copy(h_hbm.at[src_v.at[k % 4]], rows_v.at[b], gsem.at[b])
        pltpu.async_copy(s_hbm.at[pl.ds(off, K)], s_v.at[b], ssem.at[b])

    def main_wait(k):
        b = k % 2
        off = base + k * K
        pltpu.make_async_copy(h_hbm.at[src_v.at[k % 4]], rows_v.at[b], gsem.at[b]).wait()
        pltpu.make_async_copy(s_hbm.at[pl.ds(off, K)], s_v.at[b], ssem.at[b]).wait()

    def scat_start(k):
        pltpu.async_copy(rows_v.at[k % 2], acc.at[dst_v.at[k % 4]], csem.at[k % 2],
                         add=True)

    def scat_wait(k):
        pltpu.make_async_copy(rows_v.at[k % 2], acc.at[dst_v.at[k % 4]],
                              csem.at[k % 2]).wait()

    pltpu.sync_copy(zeros_hbm.at[my_rows], acc.at[my_rows])
    idx_start(0)
    idx_wait(0)
    main_start(0)
    idx_start(1)
    plsc.subcore_barrier()

    def mul_rows(b):
        def mul_row(r, carry):
            for gi in range(C // 16):
                sl = pl.ds(gi * 16, 16)
                rows_v[b, r, sl] = rows_v[b, r, sl] * s_v[b, r, sl]
            return carry
        lax.fori_loop(0, K, mul_row, 0)

    def chunk_body(g, carry):
        main_wait(g)

        @pl.when(g + 1 < NCHUNK)
        def _():
            idx_wait(g + 1)

            @pl.when(g >= 1)
            def _():
                scat_wait(g - 1)

            main_start(g + 1)

        mul_rows(g % 2)
        scat_start(g)

        @pl.when(g + 2 < NCHUNK)
        def _():
            idx_start(g + 2)

        return carry

    lax.fori_loop(0, NCHUNK, chunk_body, 0)
    scat_wait(NCHUNK - 2)
    scat_wait(NCHUNK - 1)

    plsc.subcore_barrier()
    pltpu.sync_copy(acc.at[my_rows], out_hbm.at[cid].at[my_rows])


_agg_kernel = functools.partial(
    pl.kernel,
    out_type=jax.ShapeDtypeStruct((NC, N_PAD, C), jnp.float32),
    mesh=plsc.VectorSubcoreMesh(core_axis_name="c", subcore_axis_name="s"),
    scratch_types=[
        pltpu.VMEM((4, K), jnp.int32),
        pltpu.VMEM((4, K), jnp.int32),
        pltpu.VMEM((2, K, C), jnp.float32),
        pltpu.VMEM((2, K, C), jnp.float32),
        pltpu.VMEM_SHARED((N_PAD, C), jnp.float32),
        pltpu.SemaphoreType.DMA((4,)),
        pltpu.SemaphoreType.DMA((2,)),
        pltpu.SemaphoreType.DMA((2,)),
        pltpu.SemaphoreType.DMA((2,)),
    ],
)(_agg_body)


def kernel(node_features, node_attributes, sph_harmonics, radial_basis,
           edge_index, W_pre, W_mlp1, W_mlp2, W_mlp3, W_post_int,
           W_contr, W_sc, W_post):
    f32 = jnp.float32
    pad = E_PAD - E
    src = jnp.concatenate([edge_index[0], jnp.zeros((pad,), jnp.int32)])
    dst = jnp.concatenate([edge_index[1], jnp.full((pad,), N, jnp.int32)])
    zeros = jnp.zeros((N_PAD, C), f32)

    h = pl.pallas_call(
        _pre_body,
        grid=(N // _BLK_N,),
        in_specs=[pl.BlockSpec((_BLK_N, C), lambda i: (i, 0)),
                  pl.BlockSpec((C, C), lambda i: (0, 0))],
        out_specs=pl.BlockSpec((_BLK_N, C), lambda i: (i, 0)),
        out_shape=jax.ShapeDtypeStruct((N, C), f32),
    )(node_features, W_pre)

    _last = E // _BLK_E
    s = pl.pallas_call(
        _scale_body,
        grid=(E_PAD // _BLK_E,),
        in_specs=[pl.BlockSpec((_BLK_E, RB), lambda i: (jnp.minimum(i, _last), 0)),
                  pl.BlockSpec((_BLK_E, 1), lambda i: (jnp.minimum(i, _last), 0)),
                  pl.BlockSpec((RB, H), lambda i: (0, 0)),
                  pl.BlockSpec((H, H), lambda i: (0, 0)),
                  pl.BlockSpec((H, C), lambda i: (0, 0))],
        out_specs=pl.BlockSpec((_BLK_E, C), lambda i: (i, 0)),
        out_shape=jax.ShapeDtypeStruct((E_PAD, C), f32),
    )(radial_basis, sph_harmonics, W_mlp1, W_mlp2, W_mlp3)

    partials = _agg_kernel(h, s, src, dst, zeros)

    W_sc_t = jnp.transpose(W_sc, (1, 0, 2))

    out = pl.pallas_call(
        _post_body,
        grid=(N // _BLK_N,),
        in_specs=[pl.BlockSpec((NC, _BLK_N, C), lambda i: (0, i, 0)),
                  pl.BlockSpec((_BLK_N, C), lambda i: (i, 0)),
                  pl.BlockSpec((_BLK_N, A), lambda i: (i, 0)),
                  pl.BlockSpec((C, C), lambda i: (0, 0)),
                  pl.BlockSpec((CORR, A, C), lambda i: (0, 0, 0)),
                  pl.BlockSpec((A, C, C), lambda i: (0, 0, 0)),
                  pl.BlockSpec((C, C), lambda i: (0, 0))],
        out_specs=pl.BlockSpec((_BLK_N, C), lambda i: (i, 0)),
        out_shape=jax.ShapeDtypeStruct((N, C), f32),
    )(partials[:, :N, :], node_features, node_attributes,
      W_post_int, W_contr, W_sc_t, W_post)

    return out


# speedup vs baseline: 1.2518x; 1.2518x over previous
import functools

import jax
import jax.numpy as jnp
from jax import lax
from jax.experimental import pallas as pl
from jax.experimental.pallas import tpu as pltpu
from jax.experimental.pallas import tpu_sc as plsc

N = 10000
E = 320000
C = 128
A = 10
RB = 8
H = 64
CORR = 3

NC = 2
NS = 16
NW = NC * NS
K = 64
NCHUNK = 158
EPW = K * NCHUNK
E_PAD = EPW * NW
ROWS_PER_TILE = 632
N_PAD = 10008
LAST_ROWS = N_PAD - 15 * ROWS_PER_TILE

_BLK_N = 1000
_BLK_E = 2048


def _pre_body(nf_ref, w_ref, h_ref):
    h_ref[...] = jnp.dot(nf_ref[...], w_ref[...],
                         preferred_element_type=jnp.float32)


def _mlp_body(rbt_ref, w1_ref, w2_ref, w3_ref, w_ref):
    dn = (((0,), (0,)), ((), ()))
    x = jax.nn.silu(lax.dot_general(rbt_ref[...], w1_ref[...], dn,
                                    preferred_element_type=jnp.float32))
    x = jax.nn.silu(jnp.dot(x, w2_ref[...], preferred_element_type=jnp.float32))
    w_ref[...] = jnp.dot(x, w3_ref[...], preferred_element_type=jnp.float32)


def _post_body(p_ref, nf_ref, attr_ref, wpi_ref, wc_ref, wsct_ref, wp_ref,
               out_ref):
    tm = p_ref[0] + p_ref[1]
    m = jnp.dot(tm, wpi_ref[...], preferred_element_type=jnp.float32)
    attr = attr_ref[...]
    m2 = m * m
    contracted = (jnp.dot(attr, wc_ref[0], preferred_element_type=jnp.float32) * m
                  + jnp.dot(attr, wc_ref[1], preferred_element_type=jnp.float32) * m2
                  + jnp.dot(attr, wc_ref[2], preferred_element_type=jnp.float32) * (m2 * m))
    nf = nf_ref[...]
    sc = jnp.zeros_like(m)
    for a in range(A):
        sc = sc + attr[:, a:a + 1] * jnp.dot(nf, wsct_ref[a],
                                             preferred_element_type=jnp.float32)
    out_ref[...] = jnp.dot(contracted + sc, wp_ref[...],
                           preferred_element_type=jnp.float32)


def _agg_body(h_hbm, w_hbm, sph_hbm, src_hbm, dst_hbm, zeros_hbm, out_hbm,
              idx_v, rows_v, s_v, sph_v, acc, isem, gsem, ssem, psem, csem):
    cid = lax.axis_index("c")
    sid = lax.axis_index("s")
    wid = sid * NC + cid
    base = wid * EPW

    def idx_start(k):
        off = base + k * K
        slot = idx_v.at[k % 6]
        pltpu.async_copy(src_hbm.at[pl.ds(off, K)], slot.at[0], isem.at[k % 6])
        pltpu.async_copy(dst_hbm.at[pl.ds(off, K)], slot.at[1], isem.at[k % 6])

    def idx_wait(k):
        off = base + k * K
        slot = idx_v.at[k % 6]
        pltpu.make_async_copy(src_hbm.at[pl.ds(off, K)], slot.at[0], isem.at[k % 6]).wait()
        pltpu.make_async_copy(dst_hbm.at[pl.ds(off, K)], slot.at[1], isem.at[k % 6]).wait()

    def main_start(k):
        off = base + k * K
        pltpu.async_copy(h_hbm.at[idx_v.at[k % 6].at[0]], rows_v.at[k % 4],
                         gsem.at[k % 4])
        pltpu.async_copy(w_hbm.at[pl.ds(off, K)], s_v.at[k % 2], ssem.at[k % 2])
        pltpu.async_copy(sph_hbm.at[pl.ds(off, K)],
                         sph_v.at[k % 2].at[pl.ds(0, K)], psem.at[k % 2])

    def main_wait(k):
        off = base + k * K
        pltpu.make_async_copy(h_hbm.at[idx_v.at[k % 6].at[0]], rows_v.at[k % 4],
                              gsem.at[k % 4]).wait()
        pltpu.make_async_copy(w_hbm.at[pl.ds(off, K)], s_v.at[k % 2],
                              ssem.at[k % 2]).wait()
        pltpu.make_async_copy(sph_hbm.at[pl.ds(off, K)],
                              sph_v.at[k % 2].at[pl.ds(0, K)],
                              psem.at[k % 2]).wait()

    def scat_start(k):
        pltpu.async_copy(rows_v.at[k % 4], acc.at[idx_v.at[k % 6].at[1]],
                         csem.at[k % 4], add=True)

    def scat_wait(k):
        pltpu.make_async_copy(rows_v.at[k % 4], acc.at[idx_v.at[k % 6].at[1]],
                              csem.at[k % 4]).wait()

    def mul(k):
        b4 = k % 4
        b2 = k % 2

        def mul_row(r, carry):
            ssc = sph_v[b2, pl.ds(r, 16)][0]
            for gi in range(C // 16):
                sl = pl.ds(gi * 16, 16)
                rows_v[b4, r, sl] = rows_v[b4, r, sl] * s_v[b2, r, sl] * ssc
            return carry

        lax.fori_loop(0, K, mul_row, 0)

    @pl.when(sid < NS - 1)
    def _():
        r = pl.ds(sid * ROWS_PER_TILE, ROWS_PER_TILE)
        pltpu.sync_copy(zeros_hbm.at[r], acc.at[r])

    @pl.when(sid == NS - 1)
    def _():
        r = pl.ds((NS - 1) * ROWS_PER_TILE, LAST_ROWS)
        pltpu.sync_copy(zeros_hbm.at[r], acc.at[r])

    idx_start(0)
    idx_start(1)
    idx_start(2)
    idx_wait(0)
    main_start(0)
    idx_wait(1)
    main_start(1)
    plsc.subcore_barrier()

    def chunk_body(g, carry):
        main_wait(g)
        mul(g)
        scat_start(g)

        @pl.when(g + 2 < NCHUNK)
        def _():
            idx_wait(g + 2)

            @pl.when(g >= 2)
            def _():
                scat_wait(g - 2)

            main_start(g + 2)

        @pl.when(g + 3 < NCHUNK)
        def _():
            idx_start(g + 3)

        return carry

    lax.fori_loop(0, NCHUNK, chunk_body, 0)
    scat_wait(NCHUNK - 4)
    scat_wait(NCHUNK - 3)
    scat_wait(NCHUNK - 2)
    scat_wait(NCHUNK - 1)

    plsc.subcore_barrier()

    @pl.when(sid < NS - 1)
    def _():
        r = pl.ds(sid * ROWS_PER_TILE, ROWS_PER_TILE)
        pltpu.sync_copy(acc.at[r], out_hbm.at[cid].at[r])

    @pl.when(sid == NS - 1)
    def _():
        r = pl.ds((NS - 1) * ROWS_PER_TILE, LAST_ROWS)
        pltpu.sync_copy(acc.at[r], out_hbm.at[cid].at[r])


_agg_kernel = functools.partial(
    pl.kernel,
    out_type=jax.ShapeDtypeStruct((NC, N_PAD, C), jnp.float32),
    mesh=plsc.VectorSubcoreMesh(core_axis_name="c", subcore_axis_name="s"),
    scratch_types=[
        pltpu.VMEM((6, 2, K), jnp.int32),
        pltpu.VMEM((4, K, C), jnp.float32),
        pltpu.VMEM((2, K, C), jnp.float32),
        pltpu.VMEM((2, K + 16), jnp.float32),
        pltpu.VMEM_SHARED((N_PAD, C), jnp.float32),
        pltpu.SemaphoreType.DMA((6,)),
        pltpu.SemaphoreType.DMA((4,)),
        pltpu.SemaphoreType.DMA((2,)),
        pltpu.SemaphoreType.DMA((2,)),
        pltpu.SemaphoreType.DMA((4,)),
    ],
)(_agg_body)


def kernel(node_features, node_attributes, sph_harmonics, radial_basis,
           edge_index, W_pre, W_mlp1, W_mlp2, W_mlp3, W_post_int,
           W_contr, W_sc, W_post):
    f32 = jnp.float32
    pad = E_PAD - E
    src = jnp.concatenate([edge_index[0], jnp.zeros((pad,), jnp.int32)])
    dst = jnp.concatenate([edge_index[1], jnp.full((pad,), N, jnp.int32)])
    sph_flat = jnp.concatenate([sph_harmonics.reshape(E), jnp.zeros((pad,), f32)])
    zeros = jnp.zeros((N_PAD, C), f32)

    h = pl.pallas_call(
        _pre_body,
        grid=(N // _BLK_N,),
        in_specs=[pl.BlockSpec((_BLK_N, C), lambda i: (i, 0)),
                  pl.BlockSpec((C, C), lambda i: (0, 0))],
        out_specs=pl.BlockSpec((_BLK_N, C), lambda i: (i, 0)),
        out_shape=jax.ShapeDtypeStruct((N, C), f32),
    )(node_features, W_pre)

    _last = E // _BLK_E
    w = pl.pallas_call(
        _mlp_body,
        grid=(E_PAD // _BLK_E,),
        in_specs=[pl.BlockSpec((RB, _BLK_E), lambda i: (0, jnp.minimum(i, _last))),
                  pl.BlockSpec((RB, H), lambda i: (0, 0)),
                  pl.BlockSpec((H, H), lambda i: (0, 0)),
                  pl.BlockSpec((H, C), lambda i: (0, 0))],
        out_specs=pl.BlockSpec((_BLK_E, C), lambda i: (i, 0)),
        out_shape=jax.ShapeDtypeStruct((E_PAD, C), f32),
    )(radial_basis.T, W_mlp1, W_mlp2, W_mlp3)

    partials = _agg_kernel(h, w, sph_flat, src, dst, zeros)

    W_sc_t = jnp.transpose(W_sc, (1, 0, 2))

    out = pl.pallas_call(
        _post_body,
        grid=(N // _BLK_N,),
        in_specs=[pl.BlockSpec((NC, _BLK_N, C), lambda i: (0, i, 0)),
                  pl.BlockSpec((_BLK_N, C), lambda i: (i, 0)),
                  pl.BlockSpec((_BLK_N, A), lambda i: (i, 0)),
                  pl.BlockSpec((C, C), lambda i: (0, 0)),
                  pl.BlockSpec((CORR, A, C), lambda i: (0, 0, 0)),
                  pl.BlockSpec((A, C, C), lambda i: (0, 0, 0)),
                  pl.BlockSpec((C, C), lambda i: (0, 0))],
        out_specs=pl.BlockSpec((_BLK_N, C), lambda i: (i, 0)),
        out_shape=jax.ShapeDtypeStruct((N, C), f32),
    )(partials[:, :N, :], node_features, node_attributes,
      W_post_int, W_contr, W_sc_t, W_post)

    return out


# speedup vs baseline: 1.7724x; 1.4159x over previous
import functools

import jax
import jax.numpy as jnp
from jax import lax
from jax.experimental import pallas as pl
from jax.experimental.pallas import tpu as pltpu
from jax.experimental.pallas import tpu_sc as plsc

N = 10000
E = 320000
C = 128
A = 10
RB = 8
H = 64
CORR = 3

NC = 2
NS = 16
NW = NC * NS
K = 64
NCHUNK = 158
EPW = K * NCHUNK
E_PAD = EPW * NW
ROWS_PER_TILE = 632
N_PAD = 10008
LAST_ROWS = N_PAD - 15 * ROWS_PER_TILE

_BLK_N = 1000
_BLK_E = 2048


def _pre_body(nf_ref, w_ref, h_ref):
    h_ref[...] = jnp.dot(nf_ref[...], w_ref[...],
                         preferred_element_type=jnp.float32)


def _mlp_body(rbt_ref, w1_ref, w2_ref, w3_ref, w_ref):
    dn = (((0,), (0,)), ((), ()))
    x = jax.nn.silu(lax.dot_general(rbt_ref[...], w1_ref[...], dn,
                                    preferred_element_type=jnp.float32))
    x = jax.nn.silu(jnp.dot(x, w2_ref[...], preferred_element_type=jnp.float32))
    w_ref[...] = jnp.dot(x, w3_ref[...], preferred_element_type=jnp.float32)


def _post_body(p_ref, nf_ref, attr_ref, wpi_ref, wc_ref, wsct_ref, wp_ref,
               out_ref):
    tm = p_ref[0] + p_ref[1]
    m = jnp.dot(tm, wpi_ref[...], preferred_element_type=jnp.float32)
    attr = attr_ref[...]
    m2 = m * m
    contracted = (jnp.dot(attr, wc_ref[0], preferred_element_type=jnp.float32) * m
                  + jnp.dot(attr, wc_ref[1], preferred_element_type=jnp.float32) * m2
                  + jnp.dot(attr, wc_ref[2], preferred_element_type=jnp.float32) * (m2 * m))
    nf = nf_ref[...]
    sc = jnp.zeros_like(m)
    for a in range(A):
        sc = sc + attr[:, a:a + 1] * jnp.dot(nf, wsct_ref[a],
                                             preferred_element_type=jnp.float32)
    out_ref[...] = jnp.dot(contracted + sc, wp_ref[...],
                           preferred_element_type=jnp.float32)


def _agg_body(h_hbm, w_hbm, sph_hbm, src_hbm, dst_hbm, zeros_hbm, out_hbm,
              idx_v, rows_v, msg_v, s_v, sph_v, acc, isem, gsem, ssem, psem,
              csem):
    cid = lax.axis_index("c")
    sid = lax.axis_index("s")
    wid = sid * NC + cid
    base = wid * EPW

    def idx_start(k):
        off = base + k * K
        slot = idx_v.at[k % 6]
        pltpu.async_copy(src_hbm.at[pl.ds(off, K)], slot.at[0], isem.at[k % 6])
        pltpu.async_copy(dst_hbm.at[pl.ds(off, K)], slot.at[1], isem.at[k % 6])

    def idx_wait(k):
        off = base + k * K
        slot = idx_v.at[k % 6]
        pltpu.make_async_copy(src_hbm.at[pl.ds(off, K)], slot.at[0], isem.at[k % 6]).wait()
        pltpu.make_async_copy(dst_hbm.at[pl.ds(off, K)], slot.at[1], isem.at[k % 6]).wait()

    def main_start(k):
        off = base + k * K
        pltpu.async_copy(h_hbm.at[idx_v.at[k % 6].at[0]], rows_v.at[k % 2],
                         gsem.at[k % 2])
        pltpu.async_copy(w_hbm.at[pl.ds(off, K)], s_v.at[k % 2], ssem.at[k % 2])
        pltpu.async_copy(sph_hbm.at[pl.ds(off, K)],
                         sph_v.at[k % 2].at[pl.ds(0, K)], psem.at[k % 2])

    def main_wait(k):
        off = base + k * K
        pltpu.make_async_copy(h_hbm.at[idx_v.at[k % 6].at[0]], rows_v.at[k % 2],
                              gsem.at[k % 2]).wait()
        pltpu.make_async_copy(w_hbm.at[pl.ds(off, K)], s_v.at[k % 2],
                              ssem.at[k % 2]).wait()
        pltpu.make_async_copy(sph_hbm.at[pl.ds(off, K)],
                              sph_v.at[k % 2].at[pl.ds(0, K)],
                              psem.at[k % 2]).wait()

    def scat_start(k):
        pltpu.async_copy(msg_v.at[k % 2], acc.at[idx_v.at[k % 6].at[1]],
                         csem.at[k % 2], add=True)

    def scat_wait(k):
        pltpu.make_async_copy(msg_v.at[k % 2], acc.at[idx_v.at[k % 6].at[1]],
                              csem.at[k % 2]).wait()

    def mul(k):
        b2 = k % 2

        @plsc.parallel_loop(0, K, 1, unroll=2)
        def mul_row(r):
            ssc = sph_v[b2, pl.ds(r, 16)][0]
            for gi in range(C // 16):
                sl = pl.ds(gi * 16, 16)
                msg_v[b2, r, sl] = rows_v[b2, r, sl] * s_v[b2, r, sl] * ssc

    @pl.when(sid < NS - 1)
    def _():
        r = pl.ds(sid * ROWS_PER_TILE, ROWS_PER_TILE)
        pltpu.sync_copy(zeros_hbm.at[r], acc.at[r])

    @pl.when(sid == NS - 1)
    def _():
        r = pl.ds((NS - 1) * ROWS_PER_TILE, LAST_ROWS)
        pltpu.sync_copy(zeros_hbm.at[r], acc.at[r])

    idx_start(0)
    idx_start(1)
    idx_start(2)
    idx_wait(0)
    main_start(0)
    idx_wait(1)
    main_start(1)
    plsc.subcore_barrier()

    def chunk_body(g, carry):
        main_wait(g)

        @pl.when(g >= 2)
        def _():
            scat_wait(g - 2)

        mul(g)
        scat_start(g)

        @pl.when(g + 2 < NCHUNK)
        def _():
            idx_wait(g + 2)
            main_start(g + 2)

        @pl.when(g + 3 < NCHUNK)
        def _():
            idx_start(g + 3)

        return carry

    lax.fori_loop(0, NCHUNK, chunk_body, 0)
    scat_wait(NCHUNK - 2)
    scat_wait(NCHUNK - 1)

    plsc.subcore_barrier()

    @pl.when(sid < NS - 1)
    def _():
        r = pl.ds(sid * ROWS_PER_TILE, ROWS_PER_TILE)
        pltpu.sync_copy(acc.at[r], out_hbm.at[cid].at[r])

    @pl.when(sid == NS - 1)
    def _():
        r = pl.ds((NS - 1) * ROWS_PER_TILE, LAST_ROWS)
        pltpu.sync_copy(acc.at[r], out_hbm.at[cid].at[r])


_agg_kernel = functools.partial(
    pl.kernel,
    out_type=jax.ShapeDtypeStruct((NC, N_PAD, C), jnp.float32),
    mesh=plsc.VectorSubcoreMesh(core_axis_name="c", subcore_axis_name="s"),
    scratch_types=[
        pltpu.VMEM((6, 2, K), jnp.int32),
        pltpu.VMEM((2, K, C), jnp.float32),
        pltpu.VMEM((2, K, C), jnp.float32),
        pltpu.VMEM((2, K, C), jnp.float32),
        pltpu.VMEM((2, K + 16), jnp.float32),
        pltpu.VMEM_SHARED((N_PAD, C), jnp.float32),
        pltpu.SemaphoreType.DMA((6,)),
        pltpu.SemaphoreType.DMA((2,)),
        pltpu.SemaphoreType.DMA((2,)),
        pltpu.SemaphoreType.DMA((2,)),
        pltpu.SemaphoreType.DMA((2,)),
    ],
)(_agg_body)


def kernel(node_features, node_attributes, sph_harmonics, radial_basis,
           edge_index, W_pre, W_mlp1, W_mlp2, W_mlp3, W_post_int,
           W_contr, W_sc, W_post):
    f32 = jnp.float32
    pad = E_PAD - E
    src = jnp.concatenate([edge_index[0], jnp.zeros((pad,), jnp.int32)])
    dst = jnp.concatenate([edge_index[1], jnp.full((pad,), N, jnp.int32)])
    sph_flat = jnp.concatenate([sph_harmonics.reshape(E), jnp.zeros((pad,), f32)])
    zeros = jnp.zeros((N_PAD, C), f32)

    h = pl.pallas_call(
        _pre_body,
        grid=(N // _BLK_N,),
        in_specs=[pl.BlockSpec((_BLK_N, C), lambda i: (i, 0)),
                  pl.BlockSpec((C, C), lambda i: (0, 0))],
        out_specs=pl.BlockSpec((_BLK_N, C), lambda i: (i, 0)),
        out_shape=jax.ShapeDtypeStruct((N, C), f32),
    )(node_features, W_pre)

    _last = E // _BLK_E
    w = pl.pallas_call(
        _mlp_body,
        grid=(E_PAD // _BLK_E,),
        in_specs=[pl.BlockSpec((RB, _BLK_E), lambda i: (0, jnp.minimum(i, _last))),
                  pl.BlockSpec((RB, H), lambda i: (0, 0)),
                  pl.BlockSpec((H, H), lambda i: (0, 0)),
                  pl.BlockSpec((H, C), lambda i: (0, 0))],
        out_specs=pl.BlockSpec((_BLK_E, C), lambda i: (i, 0)),
        out_shape=jax.ShapeDtypeStruct((E_PAD, C), f32),
    )(radial_basis.T, W_mlp1, W_mlp2, W_mlp3)

    partials = _agg_kernel(h, w, sph_flat, src, dst, zeros)

    W_sc_t = jnp.transpose(W_sc, (1, 0, 2))

    out = pl.pallas_call(
        _post_body,
        grid=(N // _BLK_N,),
        in_specs=[pl.BlockSpec((NC, _BLK_N, C), lambda i: (0, i, 0)),
                  pl.BlockSpec((_BLK_N, C), lambda i: (i, 0)),
                  pl.BlockSpec((_BLK_N, A), lambda i: (i, 0)),
                  pl.BlockSpec((C, C), lambda i: (0, 0)),
                  pl.BlockSpec((CORR, A, C), lambda i: (0, 0, 0)),
                  pl.BlockSpec((A, C, C), lambda i: (0, 0, 0)),
                  pl.BlockSpec((C, C), lambda i: (0, 0))],
        out_specs=pl.BlockSpec((_BLK_N, C), lambda i: (i, 0)),
        out_shape=jax.ShapeDtypeStruct((N, C), f32),
    )(partials[:, :N, :], node_features, node_attributes,
      W_post_int, W_contr, W_sc_t, W_post)

    return out


# speedup vs baseline: 2.3848x; 1.3455x over previous
import functools

import jax
import jax.numpy as jnp
from jax import lax
from jax.experimental import pallas as pl
from jax.experimental.pallas import tpu as pltpu
from jax.experimental.pallas import tpu_sc as plsc

N = 10000
E = 320000
C = 128
A = 10
RB = 8
H = 64
CORR = 3

NC = 2
NS = 16
NW = NC * NS
K = 64
NCHUNK = 158
EPW = K * NCHUNK
E_PAD = EPW * NW
ROWS_PER_TILE = 632
N_PAD = 10008
LAST_ROWS = N_PAD - 15 * ROWS_PER_TILE

_BLK_N = 1000
_BLK_E = 2048


def _pre_body(nf_ref, w_ref, h_ref):
    h_ref[...] = jnp.dot(nf_ref[...], w_ref[...],
                         preferred_element_type=jnp.float32)


def _mlp_body(rbt_ref, w1_ref, w2_ref, w3_ref, w_ref):
    dn = (((0,), (0,)), ((), ()))
    x = jax.nn.silu(lax.dot_general(rbt_ref[...], w1_ref[...], dn,
                                    preferred_element_type=jnp.float32))
    x = jax.nn.silu(jnp.dot(x, w2_ref[...], preferred_element_type=jnp.float32))
    w = jnp.dot(x, w3_ref[...], preferred_element_type=jnp.float32)
    valid = E - pl.program_id(0) * _BLK_E
    rows = lax.broadcasted_iota(jnp.int32, (_BLK_E, C), 0)
    w_ref[...] = jnp.where(rows < valid, w, 0.0)


def _post_body(p_ref, nf_ref, attr_ref, wpi_ref, wc_ref, wsct_ref, wp_ref,
               out_ref):
    tm = p_ref[0] + p_ref[1]
    m = jnp.dot(tm, wpi_ref[...], preferred_element_type=jnp.float32)
    attr = attr_ref[...]
    m2 = m * m
    contracted = (jnp.dot(attr, wc_ref[0], preferred_element_type=jnp.float32) * m
                  + jnp.dot(attr, wc_ref[1], preferred_element_type=jnp.float32) * m2
                  + jnp.dot(attr, wc_ref[2], preferred_element_type=jnp.float32) * (m2 * m))
    nf = nf_ref[...]
    sc = jnp.zeros_like(m)
    for a in range(A):
        sc = sc + attr[:, a:a + 1] * jnp.dot(nf, wsct_ref[a],
                                             preferred_element_type=jnp.float32)
    out_ref[...] = jnp.dot(contracted + sc, wp_ref[...],
                           preferred_element_type=jnp.float32)


def _agg_body(h_hbm, w_hbm, sph_hbm, src_hbm, dst_hbm, zeros_hbm, out_hbm,
              idx_v, rows_v, msg_v, s_v, sph_v, acc, isem, gsem, ssem, psem,
              csem):
    cid = lax.axis_index("c")
    sid = lax.axis_index("s")
    wid = sid * NC + cid
    base = wid * EPW

    def idx_start(k):
        off = base + k * K
        slot = idx_v.at[k % 6]
        pltpu.async_copy(src_hbm.at[pl.ds(off, K)], slot.at[0], isem.at[k % 6])
        pltpu.async_copy(dst_hbm.at[pl.ds(off, K)], slot.at[1], isem.at[k % 6])

    def idx_wait(k):
        off = base + k * K
        slot = idx_v.at[k % 6]
        pltpu.make_async_copy(src_hbm.at[pl.ds(off, K)], slot.at[0], isem.at[k % 6]).wait()
        pltpu.make_async_copy(dst_hbm.at[pl.ds(off, K)], slot.at[1], isem.at[k % 6]).wait()

    def main_start(k):
        off = base + k * K
        pltpu.async_copy(h_hbm.at[idx_v.at[k % 6].at[0]], rows_v.at[k % 2],
                         gsem.at[k % 2])
        pltpu.async_copy(w_hbm.at[pl.ds(off, K)], s_v.at[k % 2], ssem.at[k % 2])
        pltpu.async_copy(sph_hbm.at[pl.ds(off, K)],
                         sph_v.at[k % 2].at[pl.ds(0, K)], psem.at[k % 2])

    def main_wait(k):
        off = base + k * K
        pltpu.make_async_copy(h_hbm.at[idx_v.at[k % 6].at[0]], rows_v.at[k % 2],
                              gsem.at[k % 2]).wait()
        pltpu.make_async_copy(w_hbm.at[pl.ds(off, K)], s_v.at[k % 2],
                              ssem.at[k % 2]).wait()
        pltpu.make_async_copy(sph_hbm.at[pl.ds(off, K)],
                              sph_v.at[k % 2].at[pl.ds(0, K)],
                              psem.at[k % 2]).wait()

    def scat_start(k):
        pltpu.async_copy(msg_v.at[k % 2], acc.at[idx_v.at[k % 6].at[1]],
                         csem.at[k % 2], add=True)

    def scat_wait(k):
        pltpu.make_async_copy(msg_v.at[k % 2], acc.at[idx_v.at[k % 6].at[1]],
                              csem.at[k % 2]).wait()

    def mul(k):
        b2 = k % 2

        @plsc.parallel_loop(0, K, 1, unroll=2)
        def mul_row(r):
            ssc = sph_v[b2, pl.ds(r, 16)][0]
            for gi in range(C // 16):
                sl = pl.ds(gi * 16, 16)
                msg_v[b2, r, sl] = rows_v[b2, r, sl] * s_v[b2, r, sl] * ssc

    @pl.when(sid < NS - 1)
    def _():
        r = pl.ds(sid * ROWS_PER_TILE, ROWS_PER_TILE)
        pltpu.sync_copy(zeros_hbm.at[r], acc.at[r])

    @pl.when(sid == NS - 1)
    def _():
        r = pl.ds((NS - 1) * ROWS_PER_TILE, LAST_ROWS)
        pltpu.sync_copy(zeros_hbm.at[r], acc.at[r])

    idx_start(0)
    idx_start(1)
    idx_start(2)
    idx_wait(0)
    main_start(0)
    idx_wait(1)
    main_start(1)
    plsc.subcore_barrier()

    def chunk_body(g, carry):
        main_wait(g)

        @pl.when(g >= 2)
        def _():
            scat_wait(g - 2)

        mul(g)
        scat_start(g)

        @pl.when(g + 2 < NCHUNK)
        def _():
            idx_wait(g + 2)
            main_start(g + 2)

        @pl.when(g + 3 < NCHUNK)
        def _():
            idx_start(g + 3)

        return carry

    lax.fori_loop(0, NCHUNK, chunk_body, 0)
    scat_wait(NCHUNK - 2)
    scat_wait(NCHUNK - 1)

    plsc.subcore_barrier()

    @pl.when(sid < NS - 1)
    def _():
        r = pl.ds(sid * ROWS_PER_TILE, ROWS_PER_TILE)
        pltpu.sync_copy(acc.at[r], out_hbm.at[cid].at[r])

    @pl.when(sid == NS - 1)
    def _():
        r = pl.ds((NS - 1) * ROWS_PER_TILE, LAST_ROWS)
        pltpu.sync_copy(acc.at[r], out_hbm.at[cid].at[r])


_agg_kernel = functools.partial(
    pl.kernel,
    out_type=jax.ShapeDtypeStruct((NC, N_PAD, C), jnp.float32),
    mesh=plsc.VectorSubcoreMesh(core_axis_name="c", subcore_axis_name="s"),
    scratch_types=[
        pltpu.VMEM((6, 2, K), jnp.int32),
        pltpu.VMEM((2, K, C), jnp.float32),
        pltpu.VMEM((2, K, C), jnp.float32),
        pltpu.VMEM((2, K, C), jnp.float32),
        pltpu.VMEM((2, K + 16), jnp.float32),
        pltpu.VMEM_SHARED((N_PAD, C), jnp.float32),
        pltpu.SemaphoreType.DMA((6,)),
        pltpu.SemaphoreType.DMA((2,)),
        pltpu.SemaphoreType.DMA((2,)),
        pltpu.SemaphoreType.DMA((2,)),
        pltpu.SemaphoreType.DMA((2,)),
    ],
)(_agg_body)


def kernel(node_features, node_attributes, sph_harmonics, radial_basis,
           edge_index, W_pre, W_mlp1, W_mlp2, W_mlp3, W_post_int,
           W_contr, W_sc, W_post):
    f32 = jnp.float32
    pad = E_PAD - E
    spread = jnp.arange(pad, dtype=jnp.int32) % N
    src = jnp.concatenate([edge_index[0], spread])
    dst = jnp.concatenate([edge_index[1], spread])
    sph_flat = jnp.concatenate([sph_harmonics.reshape(E), jnp.zeros((pad,), f32)])
    zeros = jnp.zeros((N_PAD, C), f32)

    h = pl.pallas_call(
        _pre_body,
        grid=(N // _BLK_N,),
        in_specs=[pl.BlockSpec((_BLK_N, C), lambda i: (i, 0)),
                  pl.BlockSpec((C, C), lambda i: (0, 0))],
        out_specs=pl.BlockSpec((_BLK_N, C), lambda i: (i, 0)),
        out_shape=jax.ShapeDtypeStruct((N, C), f32),
    )(node_features, W_pre)

    _last = E // _BLK_E
    w = pl.pallas_call(
        _mlp_body,
        grid=(E_PAD // _BLK_E,),
        in_specs=[pl.BlockSpec((RB, _BLK_E), lambda i: (0, jnp.minimum(i, _last))),
                  pl.BlockSpec((RB, H), lambda i: (0, 0)),
                  pl.BlockSpec((H, H), lambda i: (0, 0)),
                  pl.BlockSpec((H, C), lambda i: (0, 0))],
        out_specs=pl.BlockSpec((_BLK_E, C), lambda i: (i, 0)),
        out_shape=jax.ShapeDtypeStruct((E_PAD, C), f32),
    )(radial_basis.T, W_mlp1, W_mlp2, W_mlp3)

    partials = _agg_kernel(h, w, sph_flat, src, dst, zeros)

    W_sc_t = jnp.transpose(W_sc, (1, 0, 2))

    out = pl.pallas_call(
        _post_body,
        grid=(N // _BLK_N,),
        in_specs=[pl.BlockSpec((NC, _BLK_N, C), lambda i: (0, i, 0)),
                  pl.BlockSpec((_BLK_N, C), lambda i: (i, 0)),
                  pl.BlockSpec((_BLK_N, A), lambda i: (i, 0)),
                  pl.BlockSpec((C, C), lambda i: (0, 0)),
                  pl.BlockSpec((CORR, A, C), lambda i: (0, 0, 0)),
                  pl.BlockSpec((A, C, C), lambda i: (0, 0, 0)),
                  pl.BlockSpec((C, C), lambda i: (0, 0))],
        out_specs=pl.BlockSpec((_BLK_N, C), lambda i: (i, 0)),
        out_shape=jax.ShapeDtypeStruct((N, C), f32),
    )(partials[:, :N, :], node_features, node_attributes,
      W_post_int, W_contr, W_sc_t, W_post)

    return out


# speedup vs baseline: 2.6407x; 1.1073x over previous
import functools

import jax
import jax.numpy as jnp
from jax import lax
from jax.experimental import pallas as pl
from jax.experimental.pallas import tpu as pltpu
from jax.experimental.pallas import tpu_sc as plsc

N = 10000
E = 320000
C = 128
A = 10
RB = 8
H = 64
CORR = 3

NC = 2
NS = 16
NW = NC * NS
K = 64
NCHUNK = 158
EPW = K * NCHUNK
E_PAD = EPW * NW
ROWS_PER_TILE = 632
N_PAD = 10008
LAST_ROWS = N_PAD - 15 * ROWS_PER_TILE

_BLK_N = 1000
_BLK_E = 4096


def _pre_body(nf_ref, w_ref, h_ref):
    h_ref[...] = jnp.dot(nf_ref[...], w_ref[...],
                         preferred_element_type=jnp.float32)


def _mlp_body(rbt_ref, w1_ref, w2_ref, w3_ref, w_ref):
    dn = (((0,), (0,)), ((), ()))
    x = jax.nn.silu(lax.dot_general(rbt_ref[...], w1_ref[...], dn,
                                    preferred_element_type=jnp.float32))
    x = jax.nn.silu(jnp.dot(x, w2_ref[...], preferred_element_type=jnp.float32))
    w = jnp.dot(x, w3_ref[...], preferred_element_type=jnp.float32)
    valid = E - pl.program_id(0) * _BLK_E
    rows = lax.broadcasted_iota(jnp.int32, (_BLK_E, C), 0)
    w_ref[...] = jnp.where(rows < valid, w, 0.0)


def _post_body(p_ref, nf_ref, attr_ref, wpi_ref, wc_ref, wsct_ref, wp_ref,
               out_ref):
    tm = p_ref[0] + p_ref[1]
    m = jnp.dot(tm, wpi_ref[...], preferred_element_type=jnp.float32)
    attr = attr_ref[...]
    m2 = m * m
    contracted = (jnp.dot(attr, wc_ref[0], preferred_element_type=jnp.float32) * m
                  + jnp.dot(attr, wc_ref[1], preferred_element_type=jnp.float32) * m2
                  + jnp.dot(attr, wc_ref[2], preferred_element_type=jnp.float32) * (m2 * m))
    nf = nf_ref[...]
    sc = jnp.zeros_like(m)
    for a in range(A):
        sc = sc + attr[:, a:a + 1] * jnp.dot(nf, wsct_ref[a],
                                             preferred_element_type=jnp.float32)
    out_ref[...] = jnp.dot(contracted + sc, wp_ref[...],
                           preferred_element_type=jnp.float32)


def _agg_body(h_hbm, w_hbm, sph_hbm, src_hbm, dst_hbm, zeros_hbm, out_hbm,
              idx_v, rows_v, msg_v, s_v, sph_v, acc, isem, gsem, ssem, psem,
              csem):
    cid = lax.axis_index("c")
    sid = lax.axis_index("s")
    wid = sid * NC + cid
    base = wid * EPW

    def idx_start(k):
        off = base + k * K
        slot = idx_v.at[k % 6]
        pltpu.async_copy(src_hbm.at[pl.ds(off, K)], slot.at[0], isem.at[k % 6])
        pltpu.async_copy(dst_hbm.at[pl.ds(off, K)], slot.at[1], isem.at[k % 6])

    def idx_wait(k):
        off = base + k * K
        slot = idx_v.at[k % 6]
        pltpu.make_async_copy(src_hbm.at[pl.ds(off, K)], slot.at[0], isem.at[k % 6]).wait()
        pltpu.make_async_copy(dst_hbm.at[pl.ds(off, K)], slot.at[1], isem.at[k % 6]).wait()

    def main_start(k):
        off = base + k * K
        pltpu.async_copy(h_hbm.at[idx_v.at[k % 6].at[0]], rows_v.at[k % 2],
                         gsem.at[k % 2])
        pltpu.async_copy(w_hbm.at[pl.ds(off, K)], s_v.at[k % 2], ssem.at[k % 2])
        pltpu.async_copy(sph_hbm.at[pl.ds(off, K)],
                         sph_v.at[k % 2].at[pl.ds(0, K)], psem.at[k % 2])

    def main_wait(k):
        off = base + k * K
        pltpu.make_async_copy(h_hbm.at[idx_v.at[k % 6].at[0]], rows_v.at[k % 2],
                              gsem.at[k % 2]).wait()
        pltpu.make_async_copy(w_hbm.at[pl.ds(off, K)], s_v.at[k % 2],
                              ssem.at[k % 2]).wait()
        pltpu.make_async_copy(sph_hbm.at[pl.ds(off, K)],
                              sph_v.at[k % 2].at[pl.ds(0, K)],
                              psem.at[k % 2]).wait()

    def scat_start(k):
        pltpu.async_copy(msg_v.at[k % 2], acc.at[idx_v.at[k % 6].at[1]],
                         csem.at[k % 2], add=True)

    def scat_wait(k):
        pltpu.make_async_copy(msg_v.at[k % 2], acc.at[idx_v.at[k % 6].at[1]],
                              csem.at[k % 2]).wait()

    def mul(k):
        b2 = k % 2

        @plsc.parallel_loop(0, K, 1, unroll=2)
        def mul_row(r):
            ssc = sph_v[b2, pl.ds(r, 16)][0]
            for gi in range(C // 16):
                sl = pl.ds(gi * 16, 16)
                msg_v[b2, r, sl] = rows_v[b2, r, sl] * s_v[b2, r, sl] * ssc

    @pl.when(sid < NS - 1)
    def _():
        r = pl.ds(sid * ROWS_PER_TILE, ROWS_PER_TILE)
        pltpu.sync_copy(zeros_hbm.at[r], acc.at[r])

    @pl.when(sid == NS - 1)
    def _():
        r = pl.ds((NS - 1) * ROWS_PER_TILE, LAST_ROWS)
        pltpu.sync_copy(zeros_hbm.at[r], acc.at[r])

    idx_start(0)
    idx_start(1)
    idx_start(2)
    idx_wait(0)
    main_start(0)
    idx_wait(1)
    main_start(1)
    plsc.subcore_barrier()

    def chunk_body(g, carry):
        main_wait(g)

        @pl.when(g >= 2)
        def _():
            scat_wait(g - 2)

        mul(g)
        scat_start(g)

        @pl.when(g + 2 < NCHUNK)
        def _():
            idx_wait(g + 2)
            main_start(g + 2)

        @pl.when(g + 3 < NCHUNK)
        def _():
            idx_start(g + 3)

        return carry

    lax.fori_loop(0, NCHUNK, chunk_body, 0)
    scat_wait(NCHUNK - 2)
    scat_wait(NCHUNK - 1)

    plsc.subcore_barrier()

    @pl.when(sid < NS - 1)
    def _():
        r = pl.ds(sid * ROWS_PER_TILE, ROWS_PER_TILE)
        pltpu.sync_copy(acc.at[r], out_hbm.at[cid].at[r])

    @pl.when(sid == NS - 1)
    def _():
        r = pl.ds((NS - 1) * ROWS_PER_TILE, LAST_ROWS)
        pltpu.sync_copy(acc.at[r], out_hbm.at[cid].at[r])


_agg_kernel = functools.partial(
    pl.kernel,
    out_type=jax.ShapeDtypeStruct((NC, N_PAD, C), jnp.float32),
    mesh=plsc.VectorSubcoreMesh(core_axis_name="c", subcore_axis_name="s"),
    scratch_types=[
        pltpu.VMEM((6, 2, K), jnp.int32),
        pltpu.VMEM((2, K, C), jnp.float32),
        pltpu.VMEM((2, K, C), jnp.float32),
        pltpu.VMEM((2, K, C), jnp.float32),
        pltpu.VMEM((2, K + 16), jnp.float32),
        pltpu.VMEM_SHARED((N_PAD, C), jnp.float32),
        pltpu.SemaphoreType.DMA((6,)),
        pltpu.SemaphoreType.DMA((2,)),
        pltpu.SemaphoreType.DMA((2,)),
        pltpu.SemaphoreType.DMA((2,)),
        pltpu.SemaphoreType.DMA((2,)),
    ],
)(_agg_body)


def kernel(node_features, node_attributes, sph_harmonics, radial_basis,
           edge_index, W_pre, W_mlp1, W_mlp2, W_mlp3, W_post_int,
           W_contr, W_sc, W_post):
    f32 = jnp.float32
    pad = E_PAD - E
    spread = jnp.arange(pad, dtype=jnp.int32) % N
    src = jnp.concatenate([edge_index[0], spread])
    dst = jnp.concatenate([edge_index[1], spread])
    sph_flat = jnp.concatenate([sph_harmonics.reshape(E), jnp.zeros((pad,), f32)])
    zeros = jnp.zeros((N_PAD, C), f32)

    h = pl.pallas_call(
        _pre_body,
        grid=(N // _BLK_N,),
        in_specs=[pl.BlockSpec((_BLK_N, C), lambda i: (i, 0)),
                  pl.BlockSpec((C, C), lambda i: (0, 0))],
        out_specs=pl.BlockSpec((_BLK_N, C), lambda i: (i, 0)),
        out_shape=jax.ShapeDtypeStruct((N, C), f32),
    )(node_features, W_pre)

    _last = E // _BLK_E
    w = pl.pallas_call(
        _mlp_body,
        grid=(E_PAD // _BLK_E,),
        in_specs=[pl.BlockSpec((RB, _BLK_E), lambda i: (0, jnp.minimum(i, _last))),
                  pl.BlockSpec((RB, H), lambda i: (0, 0)),
                  pl.BlockSpec((H, H), lambda i: (0, 0)),
                  pl.BlockSpec((H, C), lambda i: (0, 0))],
        out_specs=pl.BlockSpec((_BLK_E, C), lambda i: (i, 0)),
        out_shape=jax.ShapeDtypeStruct((E_PAD, C), f32),
    )(radial_basis.T, W_mlp1, W_mlp2, W_mlp3)

    partials = _agg_kernel(h, w, sph_flat, src, dst, zeros)

    W_sc_t = jnp.transpose(W_sc, (1, 0, 2))

    out = pl.pallas_call(
        _post_body,
        grid=(N // _BLK_N,),
        in_specs=[pl.BlockSpec((NC, _BLK_N, C), lambda i: (0, i, 0)),
                  pl.BlockSpec((_BLK_N, C), lambda i: (i, 0)),
                  pl.BlockSpec((_BLK_N, A), lambda i: (i, 0)),
                  pl.BlockSpec((C, C), lambda i: (0, 0)),
                  pl.BlockSpec((CORR, A, C), lambda i: (0, 0, 0)),
                  pl.BlockSpec((A, C, C), lambda i: (0, 0, 0)),
                  pl.BlockSpec((C, C), lambda i: (0, 0))],
        out_specs=pl.BlockSpec((_BLK_N, C), lambda i: (i, 0)),
        out_shape=jax.ShapeDtypeStruct((N, C), f32),
    )(partials, node_features, node_attributes,
      W_post_int, W_contr, W_sc_t, W_post)

    return out


# speedup vs baseline: 2.7250x; 1.0319x over previous
import functools

import jax
import jax.numpy as jnp
from jax import lax
from jax.experimental import pallas as pl
from jax.experimental.pallas import tpu as pltpu
from jax.experimental.pallas import tpu_sc as plsc

N = 10000
E = 320000
C = 128
A = 10
RB = 8
H = 64
CORR = 3

NC = 2
NS = 16
NW = NC * NS
K = 64
NCHUNK = 158
EPW = K * NCHUNK
E_PAD = EPW * NW
ROWS_PER_TILE = 632
N_PAD = 10008
LAST_ROWS = N_PAD - 15 * ROWS_PER_TILE

_BLK_N = 1000
_BLK_E = 8192


def _pre_body(nf_ref, w_ref, h_ref):
    h_ref[...] = jnp.dot(nf_ref[...], w_ref[...],
                         preferred_element_type=jnp.float32)


def _mlp_body(rbt_ref, w1_ref, w2_ref, w3_ref, w_ref):
    dn = (((0,), (0,)), ((), ()))
    x = jax.nn.silu(lax.dot_general(rbt_ref[...], w1_ref[...], dn,
                                    preferred_element_type=jnp.float32))
    x = jax.nn.silu(jnp.dot(x, w2_ref[...], preferred_element_type=jnp.float32))
    w = jnp.dot(x, w3_ref[...], preferred_element_type=jnp.float32)
    valid = E - pl.program_id(0) * _BLK_E
    rows = lax.broadcasted_iota(jnp.int32, (_BLK_E, C), 0)
    w_ref[...] = jnp.where(rows < valid, w, 0.0)


def _post_body(p_ref, nf_ref, attr_ref, wpi_ref, wc_ref, wsct_ref, wp_ref,
               out_ref):
    tm = p_ref[0] + p_ref[1]
    m = jnp.dot(tm, wpi_ref[...], preferred_element_type=jnp.float32)
    attr = attr_ref[...]
    m2 = m * m
    contracted = (jnp.dot(attr, wc_ref[0], preferred_element_type=jnp.float32) * m
                  + jnp.dot(attr, wc_ref[1], preferred_element_type=jnp.float32) * m2
                  + jnp.dot(attr, wc_ref[2], preferred_element_type=jnp.float32) * (m2 * m))
    nf = nf_ref[...]
    sc = jnp.zeros_like(m)
    for a in range(A):
        sc = sc + attr[:, a:a + 1] * jnp.dot(nf, wsct_ref[a],
                                             preferred_element_type=jnp.float32)
    out_ref[...] = jnp.dot(contracted + sc, wp_ref[...],
                           preferred_element_type=jnp.float32)


def _agg_body(h_hbm, w_hbm, sph_hbm, src_hbm, dst_hbm, zeros_hbm, out_hbm,
              idx_v, rows_v, msg_v, s_v, sph_v, acc, isem, gsem, ssem, psem,
              csem):
    cid = lax.axis_index("c")
    sid = lax.axis_index("s")
    wid = sid * NC + cid
    base = wid * EPW

    def idx_start(k):
        off = base + k * K
        slot = idx_v.at[k % 6]
        pltpu.async_copy(src_hbm.at[pl.ds(off, K)], slot.at[0], isem.at[k % 6])
        pltpu.async_copy(dst_hbm.at[pl.ds(off, K)], slot.at[1], isem.at[k % 6])

    def idx_wait(k):
        off = base + k * K
        slot = idx_v.at[k % 6]
        pltpu.make_async_copy(src_hbm.at[pl.ds(off, K)], slot.at[0], isem.at[k % 6]).wait()
        pltpu.make_async_copy(dst_hbm.at[pl.ds(off, K)], slot.at[1], isem.at[k % 6]).wait()

    def main_start(k):
        off = base + k * K
        pltpu.async_copy(h_hbm.at[idx_v.at[k % 6].at[0]], rows_v.at[k % 2],
                         gsem.at[k % 2])
        pltpu.async_copy(w_hbm.at[pl.ds(off, K)], s_v.at[k % 2], ssem.at[k % 2])
        pltpu.async_copy(sph_hbm.at[pl.ds(off, K)],
                         sph_v.at[k % 2].at[pl.ds(0, K)], psem.at[k % 2])

    def main_wait(k):
        off = base + k * K
        pltpu.make_async_copy(h_hbm.at[idx_v.at[k % 6].at[0]], rows_v.at[k % 2],
                              gsem.at[k % 2]).wait()
        pltpu.make_async_copy(w_hbm.at[pl.ds(off, K)], s_v.at[k % 2],
                              ssem.at[k % 2]).wait()
        pltpu.make_async_copy(sph_hbm.at[pl.ds(off, K)],
                              sph_v.at[k % 2].at[pl.ds(0, K)],
                              psem.at[k % 2]).wait()

    def scat_start(k):
        pltpu.async_copy(msg_v.at[k % 2], acc.at[idx_v.at[k % 6].at[1]],
                         csem.at[k % 2], add=True)

    def scat_wait(k):
        pltpu.make_async_copy(msg_v.at[k % 2], acc.at[idx_v.at[k % 6].at[1]],
                              csem.at[k % 2]).wait()

    def mul(k):
        b2 = k % 2

        @plsc.parallel_loop(0, K, 1, unroll=2)
        def mul_row(r):
            ssc = sph_v[b2, pl.ds(r, 16)][0]
            for gi in range(C // 16):
                sl = pl.ds(gi * 16, 16)
                msg_v[b2, r, sl] = rows_v[b2, r, sl] * s_v[b2, r, sl] * ssc

    @pl.when(sid < NS - 1)
    def _():
        r = pl.ds(sid * ROWS_PER_TILE, ROWS_PER_TILE)
        pltpu.sync_copy(zeros_hbm.at[r], acc.at[r])

    @pl.when(sid == NS - 1)
    def _():
        r = pl.ds((NS - 1) * ROWS_PER_TILE, LAST_ROWS)
        pltpu.sync_copy(zeros_hbm.at[r], acc.at[r])

    idx_start(0)
    idx_start(1)
    idx_start(2)
    idx_wait(0)
    main_start(0)
    idx_wait(1)
    main_start(1)
    plsc.subcore_barrier()

    def chunk_body(g, carry):
        main_wait(g)

        @pl.when(g >= 2)
        def _():
            scat_wait(g - 2)

        mul(g)
        scat_start(g)

        @pl.when(g + 2 < NCHUNK)
        def _():
            idx_wait(g + 2)
            main_start(g + 2)

        @pl.when(g + 3 < NCHUNK)
        def _():
            idx_start(g + 3)

        return carry

    lax.fori_loop(0, NCHUNK, chunk_body, 0)
    scat_wait(NCHUNK - 2)
    scat_wait(NCHUNK - 1)

    plsc.subcore_barrier()

    @pl.when(sid < NS - 1)
    def _():
        r = pl.ds(sid * ROWS_PER_TILE, ROWS_PER_TILE)
        pltpu.sync_copy(acc.at[r], out_hbm.at[cid].at[r])

    @pl.when(sid == NS - 1)
    def _():
        r = pl.ds((NS - 1) * ROWS_PER_TILE, LAST_ROWS)
        pltpu.sync_copy(acc.at[r], out_hbm.at[cid].at[r])


_agg_kernel = functools.partial(
    pl.kernel,
    out_type=jax.ShapeDtypeStruct((NC, N_PAD, C), jnp.float32),
    mesh=plsc.VectorSubcoreMesh(core_axis_name="c", subcore_axis_name="s"),
    scratch_types=[
        pltpu.VMEM((6, 2, K), jnp.int32),
        pltpu.VMEM((2, K, C), jnp.float32),
        pltpu.VMEM((2, K, C), jnp.float32),
        pltpu.VMEM((2, K, C), jnp.float32),
        pltpu.VMEM((2, K + 16), jnp.float32),
        pltpu.VMEM_SHARED((N_PAD, C), jnp.float32),
        pltpu.SemaphoreType.DMA((6,)),
        pltpu.SemaphoreType.DMA((2,)),
        pltpu.SemaphoreType.DMA((2,)),
        pltpu.SemaphoreType.DMA((2,)),
        pltpu.SemaphoreType.DMA((2,)),
    ],
)(_agg_body)


def kernel(node_features, node_attributes, sph_harmonics, radial_basis,
           edge_index, W_pre, W_mlp1, W_mlp2, W_mlp3, W_post_int,
           W_contr, W_sc, W_post):
    f32 = jnp.float32
    pad = E_PAD - E
    spread = jnp.arange(pad, dtype=jnp.int32) % N
    src = jnp.concatenate([edge_index[0], spread])
    dst = jnp.concatenate([edge_index[1], spread])
    sph_p = jnp.concatenate([sph_harmonics.reshape(E), jnp.zeros((pad,), f32)])
    zeros = jnp.zeros((N_PAD, C), f32)

    h = pl.pallas_call(
        _pre_body,
        grid=(N // _BLK_N,),
        in_specs=[pl.BlockSpec((_BLK_N, C), lambda i: (i, 0)),
                  pl.BlockSpec((C, C), lambda i: (0, 0))],
        out_specs=pl.BlockSpec((_BLK_N, C), lambda i: (i, 0)),
        out_shape=jax.ShapeDtypeStruct((N, C), f32),
    )(node_features, W_pre)

    _last = E // _BLK_E
    w = pl.pallas_call(
        _mlp_body,
        grid=(-(-E_PAD // _BLK_E),),
        in_specs=[pl.BlockSpec((RB, _BLK_E), lambda i: (0, jnp.minimum(i, _last))),
                  pl.BlockSpec((RB, H), lambda i: (0, 0)),
                  pl.BlockSpec((H, H), lambda i: (0, 0)),
                  pl.BlockSpec((H, C), lambda i: (0, 0))],
        out_specs=pl.BlockSpec((_BLK_E, C), lambda i: (i, 0)),
        out_shape=jax.ShapeDtypeStruct((E_PAD, C), f32),
    )(radial_basis.T, W_mlp1, W_mlp2, W_mlp3)

    partials = _agg_kernel(h, w, sph_p, src, dst, zeros)

    W_sc_t = jnp.transpose(W_sc, (1, 0, 2))

    out = pl.pallas_call(
        _post_body,
        grid=(N // _BLK_N,),
        in_specs=[pl.BlockSpec((NC, _BLK_N, C), lambda i: (0, i, 0)),
                  pl.BlockSpec((_BLK_N, C), lambda i: (i, 0)),
                  pl.BlockSpec((_BLK_N, A), lambda i: (i, 0)),
                  pl.BlockSpec((C, C), lambda i: (0, 0)),
                  pl.BlockSpec((CORR, A, C), lambda i: (0, 0, 0)),
                  pl.BlockSpec((A, C, C), lambda i: (0, 0, 0)),
                  pl.BlockSpec((C, C), lambda i: (0, 0))],
        out_specs=pl.BlockSpec((_BLK_N, C), lambda i: (i, 0)),
        out_shape=jax.ShapeDtypeStruct((N, C), f32),
    )(partials, node_features, node_attributes,
      W_post_int, W_contr, W_sc_t, W_post)

    return out
